# Initial kernel scaffold; baseline (speedup 1.0000x reference)
import jax
import jax.numpy as jnp
from jax.experimental import pallas as pl


def kernel(x, edge_index, edge_weight, W_rel1, b_rel1, W_root1, W_rel2, b_rel2, W_root2):
    raise NotImplementedError("write your pallas kernel here")


# speedup vs baseline: 3.1035x; 3.1035x over previous
import functools

import jax
import jax.numpy as jnp
from jax import lax
from jax.experimental import pallas as pl
from jax.experimental.pallas import tpu as pltpu
from jax.experimental.pallas import tpu_sc as plsc

N_NODES = 10000
N_PAD = 10240
D = 128
NC = 2
NS = 16
CHUNK = 128
LANES = 16
ROWS_PER_TILE = N_PAD // NS


def _lane_broadcast(vec, lane):
    idx = jnp.full((LANES, 1), lane, dtype=jnp.int32)
    dnums = lax.GatherDimensionNumbers(
        offset_dims=(), collapsed_slice_dims=(0,), start_index_map=(0,))
    return lax.gather(vec, idx, dnums, slice_sizes=(1,),
                      mode=lax.GatherScatterMode.PROMISE_IN_BOUNDS)


def _make_seg_sum(n_chunks_per_tile):
    mesh = plsc.VectorSubcoreMesh(core_axis_name="c", subcore_axis_name="s")

    @functools.partial(
        pl.kernel,
        mesh=mesh,
        out_type=jax.ShapeDtypeStruct((NC, N_PAD, D), jnp.float32),
        scratch_types=[
            pltpu.VMEM((n_chunks_per_tile, CHUNK), jnp.int32),
            pltpu.VMEM((n_chunks_per_tile, CHUNK), jnp.int32),
            pltpu.VMEM((n_chunks_per_tile, CHUNK), jnp.float32),
            pltpu.VMEM((CHUNK, D), jnp.float32),
            pltpu.VMEM_SHARED((N_PAD, D), jnp.float32),
            pltpu.SemaphoreType.DMA,
        ],
    )
    def seg_sum(x_hbm, src_hbm, dst_hbm, w_hbm, out_hbm,
                src_v, dst_v, w_v, rows_v, acc, sem):
        c = lax.axis_index("c")
        s = lax.axis_index("s")
        tile = c * NS + s

        def _zero_row(i, carry):
            for j in range(D // LANES):
                rows_v[i, pl.ds(j * LANES, LANES)] = jnp.zeros((LANES,), jnp.float32)
            return carry
        lax.fori_loop(0, CHUNK, _zero_row, 0)

        base = s * ROWS_PER_TILE
        for t in range(ROWS_PER_TILE // CHUNK):
            pltpu.sync_copy(rows_v, acc.at[pl.ds(base + t * CHUNK, CHUNK)])

        ebase = tile * n_chunks_per_tile
        pltpu.sync_copy(src_hbm.at[pl.ds(ebase, n_chunks_per_tile)], src_v)
        pltpu.sync_copy(dst_hbm.at[pl.ds(ebase, n_chunks_per_tile)], dst_v)
        pltpu.sync_copy(w_hbm.at[pl.ds(ebase, n_chunks_per_tile)], w_v)

        plsc.subcore_barrier()

        def _chunk(ci, carry):
            pltpu.async_copy(x_hbm.at[src_v.at[ci]], rows_v, sem).wait()

            def _group(g, inner):
                w16 = w_v[ci, pl.ds(g * LANES, LANES)]
                for l in range(LANES):
                    wb = _lane_broadcast(w16, l)
                    e = g * LANES + l
                    for j in range(D // LANES):
                        rows_v[e, pl.ds(j * LANES, LANES)] = (
                            rows_v[e, pl.ds(j * LANES, LANES)] * wb)
                return inner
            lax.fori_loop(0, CHUNK // LANES, _group, 0)

            pltpu.sync_copy(rows_v, acc.at[dst_v.at[ci]], add=True)
            return carry
        lax.fori_loop(0, n_chunks_per_tile, _chunk, 0)

        plsc.subcore_barrier()

        pltpu.sync_copy(acc.at[pl.ds(base, ROWS_PER_TILE)],
                        out_hbm.at[c, pl.ds(base, ROWS_PER_TILE)])

    return seg_sum


def _dense(partials, x, w_rel, b_rel, w_root, relu):
    n_rows = x.shape[0]
    nb = 10
    br = n_rows // nb

    def body(p0_r, p1_r, x_r, wrel_r, b_r, wroot_r, o_r):
        agg = p0_r[...] + p1_r[...]
        acc = jnp.dot(agg, wrel_r[...], preferred_element_type=jnp.float32)
        acc = acc + jnp.dot(x_r[...], wroot_r[...],
                            preferred_element_type=jnp.float32)
        acc = acc + b_r[...]
        if relu:
            acc = jnp.maximum(acc, 0.0)
        o_r[...] = acc

    return pl.pallas_call(
        body,
        grid=(nb,),
        in_specs=[
            pl.BlockSpec((br, D), lambda i: (i, 0)),
            pl.BlockSpec((br, D), lambda i: (i, 0)),
            pl.BlockSpec((br, D), lambda i: (i, 0)),
            pl.BlockSpec((D, D), lambda i: (0, 0)),
            pl.BlockSpec((1, D), lambda i: (0, 0)),
            pl.BlockSpec((D, D), lambda i: (0, 0)),
        ],
        out_specs=pl.BlockSpec((br, D), lambda i: (i, 0)),
        out_shape=jax.ShapeDtypeStruct((n_rows, D), jnp.float32),
    )(partials[0], partials[1], x, w_rel, b_rel.reshape(1, D), w_root)


def kernel(x, edge_index, edge_weight, W_rel1, b_rel1, W_root1,
           W_rel2, b_rel2, W_root2):
    n_edges = edge_weight.shape[0]
    src = edge_index[0].astype(jnp.int32)
    dst = edge_index[1].astype(jnp.int32)
    w = edge_weight.astype(jnp.float32)

    per_round = NC * NS * CHUNK
    n_chunks_per_tile = -(-n_edges // per_round)
    n_chunks_per_tile += (-n_chunks_per_tile) % 8
    pad = n_chunks_per_tile * per_round - n_edges
    if pad:
        src = jnp.concatenate([src, jnp.zeros((pad,), jnp.int32)])
        dst = jnp.concatenate([dst, jnp.zeros((pad,), jnp.int32)])
        w = jnp.concatenate([w, jnp.zeros((pad,), jnp.float32)])

    src2 = src.reshape(NC * NS * n_chunks_per_tile, CHUNK)
    dst2 = dst.reshape(NC * NS * n_chunks_per_tile, CHUNK)
    w2 = w.reshape(NC * NS * n_chunks_per_tile, CHUNK)

    seg_sum = _make_seg_sum(n_chunks_per_tile)

    p1 = seg_sum(x, src2, dst2, w2)[:, :N_NODES]
    h = _dense(p1, x, W_rel1, b_rel1, W_root1, relu=True)
    p2 = seg_sum(h, src2, dst2, w2)[:, :N_NODES]
    out = _dense(p2, h, W_rel2, b_rel2, W_root2, relu=False)
    return out
